# Initial kernel scaffold; baseline (speedup 1.0000x reference)
import jax
import jax.numpy as jnp
from jax.experimental import pallas as pl


def kernel(sentence, rois, ridx, conv_w, conv_b, w1, b1, wc, bc, w2, b2, wb, bb):
    raise NotImplementedError("write your pallas kernel here")


# speedup vs baseline: 5.2672x; 5.2672x over previous
import jax
import jax.numpy as jnp
from jax import lax
from jax.experimental import pallas as pl
from jax.experimental.pallas import tpu as pltpu

_B, _L, _D = 64, 512, 300
_F, _K, _P = 256, 3, 8
_C = 18
_NROI = 2048
_FF = _F * _P
_NH = (_C + 1) * 3
_WIN = 40
_RPB = 64
_MIB = 1024 * 1024


def _prep_body(w1_ref, wc_ref, b1_ref, bc_ref, w2_ref, wb_ref, b2_ref, bb_ref,
               wout_ref, bout_ref):
    a1 = jnp.dot(w1_ref[...], wc_ref[...], preferred_element_type=jnp.float32)
    a2 = jnp.dot(w2_ref[...], wb_ref[...], preferred_element_type=jnp.float32)
    row = lax.broadcasted_iota(jnp.int32, (_F, _FF), 0)
    col = lax.broadcasted_iota(jnp.int32, (_F, _FF), 1)
    fold = (col // _P == row).astype(jnp.float32)
    w1e = jnp.dot(fold, a1, preferred_element_type=jnp.float32)
    w2e = jnp.dot(fold, a2, preferred_element_type=jnp.float32)
    wout_ref[...] = jnp.concatenate([w1e, w2e], axis=1)
    bv1 = jnp.dot(b1_ref[...], wc_ref[...], preferred_element_type=jnp.float32) + bc_ref[...]
    bv2 = jnp.dot(b2_ref[...], wb_ref[...], preferred_element_type=jnp.float32) + bb_ref[...]
    bout_ref[...] = jnp.concatenate([bv1, bv2], axis=1)


def _conv_body(s_ref, wt_ref, cb_ref, o_ref):
    s = s_ref[0]
    y0 = jnp.dot(s, wt_ref[0], preferred_element_type=jnp.float32)
    y1 = jnp.dot(s, wt_ref[1], preferred_element_type=jnp.float32)
    y2 = jnp.dot(s, wt_ref[2], preferred_element_type=jnp.float32)
    z = jnp.zeros((1, _F), jnp.float32)
    f = (jnp.concatenate([z, y0[:-1]], axis=0) + y1
         + jnp.concatenate([y2[1:], z], axis=0) + cb_ref[...])
    o_ref[0] = jnp.maximum(f, 0.0)


def _roi_body(rb_ref, off_ref, wid_ref, feat_hbm, weff_ref, beff_ref, out_ref,
              fbuf, tile, sem):
    i = pl.program_id(0)

    @pl.when(i == 0)
    def _():
        cp = pltpu.make_async_copy(feat_hbm, fbuf, sem)
        cp.start()
        cp.wait()

    for mi in range(_RPB):
        k = i * _RPB + mi
        rb = pl.multiple_of(rb_ref[k], 8)
        win = fbuf[pl.ds(rb, _WIN), :]
        o = off_ref[k]
        w = wid_ref[k]
        io = lax.broadcasted_iota(jnp.int32, (_WIN, _F), 0)
        msk = (io >= o) & (io < o + w)
        pooled = jnp.max(jnp.where(msk, win, jnp.float32(-1e30)), axis=0,
                         keepdims=True)
        tile[mi:mi + 1, :] = pooled

    out_ref[...] = (jnp.dot(tile[...], weff_ref[...],
                            preferred_element_type=jnp.float32) + beff_ref[...])


def kernel(sentence, rois, ridx, conv_w, conv_b, w1, b1, wc, bc, w2, b2, wb, bb):
    s2 = sentence.reshape(_B, _L, _D)
    wt = conv_w[:, 0].transpose(1, 2, 0)

    feat = pl.pallas_call(
        _conv_body,
        out_shape=jax.ShapeDtypeStruct((_B, _L, _F), jnp.float32),
        grid=(_B,),
        in_specs=[
            pl.BlockSpec((1, _L, _D), lambda i: (i, 0, 0)),
            pl.BlockSpec((_K, _D, _F), lambda i: (0, 0, 0)),
            pl.BlockSpec((1, _F), lambda i: (0, 0)),
        ],
        out_specs=pl.BlockSpec((1, _L, _F), lambda i: (i, 0, 0)),
        compiler_params=pltpu.CompilerParams(
            dimension_semantics=("parallel",),
        ),
        name="conv_relu",
    )(s2, wt, conv_b.reshape(1, _F))

    weff, beff = pl.pallas_call(
        _prep_body,
        out_shape=[
            jax.ShapeDtypeStruct((_F, _NH), jnp.float32),
            jax.ShapeDtypeStruct((1, _NH), jnp.float32),
        ],
        compiler_params=pltpu.CompilerParams(
            vmem_limit_bytes=52 * _MIB,
        ),
        name="head_weight_fold",
    )(w1, wc, b1.reshape(1, _FF), bc.reshape(1, _C + 1),
      w2, wb, b2.reshape(1, _FF), bb.reshape(1, 2 * (_C + 1)))

    starts = rois[:, 0]
    width = rois[:, 1] - starts
    al = jnp.clip((starts // 8) * 8, 0, _L - _WIN)
    rbase = (ridx.astype(jnp.int32) * _L + al).astype(jnp.int32)
    off = (starts - al).astype(jnp.int32)

    out = pl.pallas_call(
        _roi_body,
        out_shape=jax.ShapeDtypeStruct((_NROI, _NH), jnp.float32),
        grid=(_NROI // _RPB,),
        in_specs=[
            pl.BlockSpec(memory_space=pltpu.SMEM),
            pl.BlockSpec(memory_space=pltpu.SMEM),
            pl.BlockSpec(memory_space=pltpu.SMEM),
            pl.BlockSpec(memory_space=pl.ANY),
            pl.BlockSpec((_F, _NH), lambda i: (0, 0)),
            pl.BlockSpec((1, _NH), lambda i: (0, 0)),
        ],
        out_specs=pl.BlockSpec((_RPB, _NH), lambda i: (i, 0)),
        scratch_shapes=[
            pltpu.VMEM((_B * _L, _F), jnp.float32),
            pltpu.VMEM((_RPB, _F), jnp.float32),
            pltpu.SemaphoreType.DMA,
        ],
        compiler_params=pltpu.CompilerParams(
            dimension_semantics=("arbitrary",),
            vmem_limit_bytes=44 * _MIB,
        ),
        name="roi_pool_heads",
    )(rbase, off, width.astype(jnp.int32), feat.reshape(_B * _L, _F),
      weff, beff)

    cls_score = out[:, :_C + 1]
    bbox = out[:, _C + 1:].reshape(_NROI, _C + 1, 2)
    return cls_score, bbox
